# baseline (speedup 1.0000x reference)
import dataclasses

import jax
import jax.numpy as jnp
from jax import lax
from jax.experimental import pallas as pl
from jax.experimental.pallas import tpu as pltpu
from jax.experimental.pallas import tpu_sc as plsc

N_DEV = 10000
N_CAND = 10000
E = 160000
D = 256
H = 4
C = 256
HC = H * C
CH = C // 2

NPAD = 10240
BR = 512

NW = 32
EPAD = 163840
ECH_W = EPAD // NW
ECH_T = EPAD // 16
EB = 1024
NB2 = ECH_W // EB
NB3 = ECH_T // EB
NGB = EB // 16
NROWS_T = NPAD // 16


def _leaky(x, slope):
    return jnp.where(x >= 0, x, slope * x)


def _sc_params():
    cp = pltpu.CompilerParams()
    if "needs_layout_passes" in pltpu.CompilerParams.__dataclass_fields__:
        cp = dataclasses.replace(cp, needs_layout_passes=False)
    return cp


def _proj_body(dev_ref, cand_ref, Wl_ref, bl_ref, Wr_ref, br_ref, Wres_ref,
               xl_ref, xr_ref, res_ref):
    xl_ref[...] = (
        jnp.dot(dev_ref[...], Wl_ref[...], preferred_element_type=jnp.float32)
        + bl_ref[...]
    )
    xr_ref[...] = (
        jnp.dot(cand_ref[...], Wr_ref[...], preferred_element_type=jnp.float32)
        + br_ref[...]
    )
    res_ref[...] = jnp.dot(cand_ref[...], Wres_ref[...],
                           preferred_element_type=jnp.float32)


def _proj(dev, cand, Wl, bl, Wr, br, Wres):
    return pl.pallas_call(
        _proj_body,
        grid=(NPAD // BR,),
        in_specs=[
            pl.BlockSpec((BR, D), lambda i: (i, 0)),
            pl.BlockSpec((BR, D), lambda i: (i, 0)),
            pl.BlockSpec((D, HC), lambda i: (0, 0)),
            pl.BlockSpec((1, HC), lambda i: (0, 0)),
            pl.BlockSpec((D, HC), lambda i: (0, 0)),
            pl.BlockSpec((1, HC), lambda i: (0, 0)),
            pl.BlockSpec((D, C), lambda i: (0, 0)),
        ],
        out_specs=[
            pl.BlockSpec((BR, HC), lambda i: (i, 0)),
            pl.BlockSpec((BR, HC), lambda i: (i, 0)),
            pl.BlockSpec((BR, C), lambda i: (i, 0)),
        ],
        out_shape=[
            jax.ShapeDtypeStruct((NPAD, HC), jnp.float32),
            jax.ShapeDtypeStruct((NPAD, HC), jnp.float32),
            jax.ShapeDtypeStruct((NPAD, C), jnp.float32),
        ],
    )(dev, cand, Wl, bl, Wr, br, Wres)


def _p2_body(xl_hbm, xr_hbm, src_hbm, dst_hbm, att_hbm, zch_hbm,
             ex_hbm, den_hbm,
             srcb, dstb, exb, xlb, xrb, att_v, row_v, sbuf, dbuf, sem0,
             den_s):
    cid = lax.axis_index("c")
    sid = lax.axis_index("s")
    w = sid * 2 + cid
    base = w * ECH_W
    pltpu.sync_copy(att_hbm, att_v)
    pltpu.sync_copy(zch_hbm.at[pl.ds(sid * NROWS_T, NROWS_T)],
                    den_s.at[pl.ds(sid * NROWS_T, NROWS_T)])

    zeros16 = jnp.zeros((16,), jnp.float32)

    @pl.loop(0, 16)
    def _(i):
        for c8 in range(8):
            row_v[i, pl.ds(c8 * 16, 16)] = zeros16

    plsc.subcore_barrier()

    iota = lax.iota(jnp.int32, 16)

    @pl.loop(0, NB2)
    def _(b):
        bb = base + b * EB
        pltpu.sync_copy(src_hbm.at[pl.ds(bb, EB)], srcb)
        pltpu.sync_copy(dst_hbm.at[pl.ds(bb, EB)], dstb)

        @pl.loop(0, NGB)
        def _(g):
            sbuf[...] = srcb[pl.ds(g * 16, 16)]
            dbuf[...] = dstb[pl.ds(g * 16, 16)]
            pltpu.async_copy(xl_hbm.at[sbuf], xlb, sem0).wait()
            pltpu.async_copy(xr_hbm.at[dbuf], xrb, sem0).wait()
            for h in range(H):
                def cbody(cc, acc, h=h):
                    for p in range(2):
                        sub = jnp.full((16,), p * 4 + h, jnp.int32)
                        ccv = jnp.full((16,), cc, jnp.int32)
                        a = plsc.load_gather(xlb, [iota, sub, ccv])
                        bv = plsc.load_gather(xrb, [iota, sub, ccv])
                        av = plsc.load_gather(
                            att_v,
                            [jnp.full((16,), 0, jnp.int32)
                             + ((p * 4 + h) * 128 + cc)])
                        s = a + bv
                        acc = acc + jnp.maximum(s, 0.2 * s) * av
                    return acc

                acch = lax.fori_loop(0, 128, cbody, zeros16)
                exh = jnp.exp(acch)
                exb[pl.ds(h * EB + g * 16, 16)] = exh
                plsc.store_scatter(row_v,
                                   [iota, jnp.full((16,), h, jnp.int32)],
                                   exh)
            pltpu.sync_copy(row_v, den_s.at[dbuf], add=True)

        for h in range(H):
            pltpu.sync_copy(exb.at[pl.ds(h * EB, EB)],
                            ex_hbm.at[pl.ds(h * EPAD + bb, EB)])

    plsc.subcore_barrier()
    pltpu.sync_copy(den_s.at[pl.ds(sid * NROWS_T, NROWS_T)],
                    den_hbm.at[pl.ds(cid * NPAD + sid * NROWS_T, NROWS_T)])


def _p2(xl3, xr3, src, dst, attp, zch):
    mesh = plsc.VectorSubcoreMesh(core_axis_name="c", subcore_axis_name="s")
    k = pl.kernel(
        _p2_body,
        out_type=[
            jax.ShapeDtypeStruct((H * EPAD,), jnp.float32),
            jax.ShapeDtypeStruct((2 * NPAD, 128), jnp.float32),
        ],
        mesh=mesh,
        scratch_types=[
            pltpu.VMEM((EB,), jnp.int32),
            pltpu.VMEM((EB,), jnp.int32),
            pltpu.VMEM((H * EB,), jnp.float32),
            pltpu.VMEM((16, 8, 128), jnp.float32),
            pltpu.VMEM((16, 8, 128), jnp.float32),
            pltpu.VMEM((HC,), jnp.float32),
            pltpu.VMEM((16, 128), jnp.float32),
            pltpu.VMEM((16,), jnp.int32),
            pltpu.VMEM((16,), jnp.int32),
            pltpu.SemaphoreType.DMA,
            pltpu.VMEM_SHARED((NPAD, 128), jnp.float32),
        ],
        compiler_params=_sc_params(),
    )
    return k(xl3, xr3, src, dst, attp, zch)


def _inv_body(den_ref, inv_ref):
    inv_ref[...] = 1.0 / (den_ref[0] + den_ref[1] + 1e-16)


def _inv(den):
    return pl.pallas_call(
        _inv_body,
        in_specs=[pl.BlockSpec((2, NPAD, 128), lambda: (0, 0, 0))],
        out_specs=pl.BlockSpec((NPAD, 128), lambda: (0, 0)),
        out_shape=jax.ShapeDtypeStruct((NPAD, 128), jnp.float32),
    )(den.reshape(2, NPAD, 128))


def _p3_body(xl2_hbm, src_hbm, dst_hbm, ex_hbm, inv_hbm, zch_hbm,
             agg_hbm,
             srcb, dstb, exb, rowb, invb, wb, outb, sbuf, dbuf, sem0,
             acc_s):
    cid = lax.axis_index("c")
    sid = lax.axis_index("s")
    rstart = sid * NROWS_T
    pltpu.sync_copy(zch_hbm.at[pl.ds(rstart, NROWS_T)],
                    acc_s.at[pl.ds(rstart, NROWS_T)])

    plsc.subcore_barrier()

    iota = lax.iota(jnp.int32, 16)

    @pl.loop(0, NB3)
    def _(b):
        bb = sid * ECH_T + b * EB
        pltpu.sync_copy(src_hbm.at[pl.ds(bb, EB)], srcb)
        pltpu.sync_copy(dst_hbm.at[pl.ds(bb, EB)], dstb)
        for h in range(H):
            pltpu.sync_copy(ex_hbm.at[pl.ds(h * EPAD + bb, EB)],
                            exb.at[pl.ds(h * EB, EB)])

        @pl.loop(0, NGB)
        def _(g):
            sbuf[...] = srcb[pl.ds(g * 16, 16)] * 2 + cid
            dbuf[...] = dstb[pl.ds(g * 16, 16)]
            pltpu.async_copy(xl2_hbm.at[sbuf], rowb, sem0).wait()
            pltpu.async_copy(inv_hbm.at[dbuf], invb, sem0).wait()
            for h in range(H):
                hv = jnp.full((16,), h, jnp.int32)
                exh = exb[pl.ds(h * EB + g * 16, 16)]
                invh = plsc.load_gather(invb, [iota, hv])
                plsc.store_scatter(wb, [iota, hv], exh * invh * 0.25)

            @pl.loop(0, 16)
            def _(j):
                jv = jnp.full((16,), j, jnp.int32)
                ws = [plsc.load_gather(wb,
                                       [jv, jnp.full((16,), h, jnp.int32)])
                      for h in range(H)]
                for c8 in range(CH // 16):
                    cv = c8 * 16 + iota
                    acc = ws[0] * plsc.load_gather(
                        rowb, [jv, jnp.full((16,), 0, jnp.int32), cv])
                    for h in range(1, H):
                        acc = acc + ws[h] * plsc.load_gather(
                            rowb, [jv, jnp.full((16,), h, jnp.int32), cv])
                    plsc.store_scatter(outb, [jv, cv], acc)

            pltpu.sync_copy(outb, acc_s.at[dbuf], add=True)

    plsc.subcore_barrier()
    pltpu.sync_copy(acc_s.at[pl.ds(rstart, NROWS_T)],
                    agg_hbm.at[pl.ds(cid * NPAD + rstart, NROWS_T)])


def _p3(xl2, src, dst, ex, inv, zch):
    mesh = plsc.VectorSubcoreMesh(core_axis_name="c", subcore_axis_name="s")
    k = pl.kernel(
        _p3_body,
        out_type=jax.ShapeDtypeStruct((2 * NPAD, 128), jnp.float32),
        mesh=mesh,
        scratch_types=[
            pltpu.VMEM((EB,), jnp.int32),
            pltpu.VMEM((EB,), jnp.int32),
            pltpu.VMEM((H * EB,), jnp.float32),
            pltpu.VMEM((16, 4, 128), jnp.float32),
            pltpu.VMEM((16, 128), jnp.float32),
            pltpu.VMEM((16, 16), jnp.float32),
            pltpu.VMEM((16, 128), jnp.float32),
            pltpu.VMEM((16,), jnp.int32),
            pltpu.VMEM((16,), jnp.int32),
            pltpu.SemaphoreType.DMA,
            pltpu.VMEM_SHARED((NPAD, 128), jnp.float32),
        ],
        compiler_params=_sc_params(),
    )
    return k(xl2, src, dst, ex, inv, zch)


def _post_body(agg_ref, res_ref, cb_ref, g_ref, b_ref, Wout_ref, bout_ref,
               out_ref):
    x = (jnp.concatenate([agg_ref[0], agg_ref[1]], axis=-1)
         + res_ref[...] + cb_ref[...])
    mu = jnp.mean(x, axis=-1, keepdims=True)
    var = jnp.mean((x - mu) ** 2, axis=-1, keepdims=True)
    xn = (x - mu) * jax.lax.rsqrt(var + 1e-5) * g_ref[...] + b_ref[...]
    act = _leaky(xn, 0.01)
    out_ref[...] = (
        jnp.dot(act, Wout_ref[...], preferred_element_type=jnp.float32)
        + bout_ref[...]
    )


def _post(agg, res, conv_bias, ln_gamma, ln_beta, Wout, bout):
    return pl.pallas_call(
        _post_body,
        grid=(NPAD // BR,),
        in_specs=[
            pl.BlockSpec((2, BR, CH), lambda i: (0, i, 0)),
            pl.BlockSpec((BR, C), lambda i: (i, 0)),
            pl.BlockSpec((1, C), lambda i: (0, 0)),
            pl.BlockSpec((1, C), lambda i: (0, 0)),
            pl.BlockSpec((1, C), lambda i: (0, 0)),
            pl.BlockSpec((C, C), lambda i: (0, 0)),
            pl.BlockSpec((1, C), lambda i: (0, 0)),
        ],
        out_specs=pl.BlockSpec((BR, C), lambda i: (i, 0)),
        out_shape=jax.ShapeDtypeStruct((NPAD, C), jnp.float32),
    )(agg, res, conv_bias, ln_gamma, ln_beta, Wout, bout)


def kernel(device_embeddings, candidate_embedding, edge_index, Wl, bl, Wr, br,
           att, Wres, conv_bias, ln_gamma, ln_beta, Wout, bout):
    Wl_p = Wl.reshape(D, H, 2, CH).transpose(0, 2, 1, 3).reshape(D, HC)
    Wr_p = Wr.reshape(D, H, 2, CH).transpose(0, 2, 1, 3).reshape(D, HC)
    bl_p = bl.reshape(H, 2, CH).transpose(1, 0, 2).reshape(HC)
    br_p = br.reshape(H, 2, CH).transpose(1, 0, 2).reshape(HC)
    att_p = att.reshape(H, 2, CH).transpose(1, 0, 2).reshape(HC)

    dev_p = jnp.pad(device_embeddings, ((0, NPAD - N_DEV), (0, 0)))
    cand_p = jnp.pad(candidate_embedding, ((0, NPAD - N_CAND), (0, 0)))
    xl, xr, res = _proj(dev_p, cand_p, Wl_p, bl_p[None, :], Wr_p,
                        br_p[None, :], Wres)

    pad_idx = jnp.full((EPAD - E,), NPAD - 1, jnp.int32)
    src = jnp.concatenate([edge_index[0], pad_idx])
    dst = jnp.concatenate([edge_index[1], pad_idx])

    zch = jnp.zeros((NPAD, 128), jnp.float32)

    ex, den = _p2(xl.reshape(NPAD, 8, 128), xr.reshape(NPAD, 8, 128),
                  src, dst, att_p, zch)
    inv = _inv(den)
    agg = _p3(xl.reshape(NPAD * 2, 4, 128), src, dst, ex, inv, zch)
    agg = agg.reshape(2, NPAD, CH)

    out = _post(agg, res, conv_bias[None, :], ln_gamma[None, :],
                ln_beta[None, :], Wout, bout[None, :])
    return out[:N_CAND]


# speedup vs baseline: 1.0933x; 1.0933x over previous
import dataclasses

import jax
import jax.numpy as jnp
from jax import lax
from jax.experimental import pallas as pl
from jax.experimental.pallas import tpu as pltpu
from jax.experimental.pallas import tpu_sc as plsc

N_DEV = 10000
N_CAND = 10000
E = 160000
D = 256
H = 4
C = 256
HC = H * C
CH = C // 2

NPAD = 10240
BR = 512

NW = 32
EPAD = 163840
ECH_W = EPAD // NW
ECH_T = EPAD // 16
EB = 1024
NB2 = ECH_W // EB
NB3 = ECH_T // EB
NGB = EB // 16
NROWS_T = NPAD // 16


def _leaky(x, slope):
    return jnp.where(x >= 0, x, slope * x)


def _sc_params():
    cp = pltpu.CompilerParams()
    if "needs_layout_passes" in pltpu.CompilerParams.__dataclass_fields__:
        cp = dataclasses.replace(cp, needs_layout_passes=False)
    return cp


def _proj_body(dev_ref, cand_ref, Wl_ref, bl_ref, Wr_ref, br_ref, Wres_ref,
               xl_ref, xr_ref, res_ref):
    xl_ref[...] = (
        jnp.dot(dev_ref[...], Wl_ref[...], preferred_element_type=jnp.float32)
        + bl_ref[...]
    )
    xr_ref[...] = (
        jnp.dot(cand_ref[...], Wr_ref[...], preferred_element_type=jnp.float32)
        + br_ref[...]
    )
    res_ref[...] = jnp.dot(cand_ref[...], Wres_ref[...],
                           preferred_element_type=jnp.float32)


def _proj(dev, cand, Wl, bl, Wr, br, Wres):
    return pl.pallas_call(
        _proj_body,
        grid=(NPAD // BR,),
        in_specs=[
            pl.BlockSpec((BR, D), lambda i: (i, 0)),
            pl.BlockSpec((BR, D), lambda i: (i, 0)),
            pl.BlockSpec((D, HC), lambda i: (0, 0)),
            pl.BlockSpec((1, HC), lambda i: (0, 0)),
            pl.BlockSpec((D, HC), lambda i: (0, 0)),
            pl.BlockSpec((1, HC), lambda i: (0, 0)),
            pl.BlockSpec((D, C), lambda i: (0, 0)),
        ],
        out_specs=[
            pl.BlockSpec((BR, HC), lambda i: (i, 0)),
            pl.BlockSpec((BR, HC), lambda i: (i, 0)),
            pl.BlockSpec((BR, C), lambda i: (i, 0)),
        ],
        out_shape=[
            jax.ShapeDtypeStruct((NPAD, HC), jnp.float32),
            jax.ShapeDtypeStruct((NPAD, HC), jnp.float32),
            jax.ShapeDtypeStruct((NPAD, C), jnp.float32),
        ],
    )(dev, cand, Wl, bl, Wr, br, Wres)


def _p2_body(xl_hbm, xr_hbm, src_hbm, dst_hbm, att_hbm, zch_hbm,
             ex_hbm, den_hbm,
             srcb, dstb, exb, xlb, xrb, att_v, row_v, sbuf, dbuf, sem0,
             sem1, den_s):
    cid = lax.axis_index("c")
    sid = lax.axis_index("s")
    w = sid * 2 + cid
    base = w * ECH_W
    pltpu.sync_copy(att_hbm, att_v)
    pltpu.sync_copy(zch_hbm.at[pl.ds(sid * NROWS_T, NROWS_T)],
                    den_s.at[pl.ds(sid * NROWS_T, NROWS_T)])

    zeros16 = jnp.zeros((16,), jnp.float32)

    @pl.loop(0, 16)
    def _(i):
        for c8 in range(8):
            row_v[i, pl.ds(c8 * 16, 16)] = zeros16

    plsc.subcore_barrier()

    iota = lax.iota(jnp.int32, 16)

    @pl.loop(0, NB2)
    def _(b):
        bb = base + b * EB
        pltpu.sync_copy(src_hbm.at[pl.ds(bb, EB)], srcb)
        pltpu.sync_copy(dst_hbm.at[pl.ds(bb, EB)], dstb)

        @pl.loop(0, NGB)
        def _(g):
            sbuf[...] = srcb[pl.ds(g * 16, 16)]
            dbuf[...] = dstb[pl.ds(g * 16, 16)]
            cpa = pltpu.async_copy(xl_hbm.at[sbuf], xlb, sem0)
            cpb = pltpu.async_copy(xr_hbm.at[dbuf], xrb, sem1)
            cpa.wait()
            cpb.wait()
            for h in range(H):
                def cbody(cc, acc, h=h):
                    for p in range(2):
                        sub = jnp.full((16,), p * 4 + h, jnp.int32)
                        ccv = jnp.full((16,), cc, jnp.int32)
                        a = plsc.load_gather(xlb, [iota, sub, ccv])
                        bv = plsc.load_gather(xrb, [iota, sub, ccv])
                        av = plsc.load_gather(
                            att_v,
                            [jnp.full((16,), 0, jnp.int32)
                             + ((p * 4 + h) * 128 + cc)])
                        s = a + bv
                        acc = acc + jnp.maximum(s, 0.2 * s) * av
                    return acc

                acch = lax.fori_loop(0, 128, cbody, zeros16)
                exh = jnp.exp(acch)
                exb[pl.ds(h * EB + g * 16, 16)] = exh
                plsc.store_scatter(row_v,
                                   [iota, jnp.full((16,), h, jnp.int32)],
                                   exh)
            pltpu.sync_copy(row_v, den_s.at[dbuf], add=True)

        for h in range(H):
            pltpu.sync_copy(exb.at[pl.ds(h * EB, EB)],
                            ex_hbm.at[pl.ds(h * EPAD + bb, EB)])

    plsc.subcore_barrier()
    pltpu.sync_copy(den_s.at[pl.ds(sid * NROWS_T, NROWS_T)],
                    den_hbm.at[pl.ds(cid * NPAD + sid * NROWS_T, NROWS_T)])


def _p2(xl3, xr3, src, dst, attp, zch):
    mesh = plsc.VectorSubcoreMesh(core_axis_name="c", subcore_axis_name="s")
    k = pl.kernel(
        _p2_body,
        out_type=[
            jax.ShapeDtypeStruct((H * EPAD,), jnp.float32),
            jax.ShapeDtypeStruct((2 * NPAD, 128), jnp.float32),
        ],
        mesh=mesh,
        scratch_types=[
            pltpu.VMEM((EB,), jnp.int32),
            pltpu.VMEM((EB,), jnp.int32),
            pltpu.VMEM((H * EB,), jnp.float32),
            pltpu.VMEM((16, 8, 128), jnp.float32),
            pltpu.VMEM((16, 8, 128), jnp.float32),
            pltpu.VMEM((HC,), jnp.float32),
            pltpu.VMEM((16, 128), jnp.float32),
            pltpu.VMEM((16,), jnp.int32),
            pltpu.VMEM((16,), jnp.int32),
            pltpu.SemaphoreType.DMA,
            pltpu.SemaphoreType.DMA,
            pltpu.VMEM_SHARED((NPAD, 128), jnp.float32),
        ],
        compiler_params=_sc_params(),
    )
    return k(xl3, xr3, src, dst, attp, zch)


def _inv_body(den_ref, inv_ref):
    inv_ref[...] = 1.0 / (den_ref[0] + den_ref[1] + 1e-16)


def _inv(den):
    return pl.pallas_call(
        _inv_body,
        in_specs=[pl.BlockSpec((2, NPAD, 128), lambda: (0, 0, 0))],
        out_specs=pl.BlockSpec((NPAD, 128), lambda: (0, 0)),
        out_shape=jax.ShapeDtypeStruct((NPAD, 128), jnp.float32),
    )(den.reshape(2, NPAD, 128))


def _p3_body(xl2_hbm, src_hbm, dst_hbm, ex_hbm, inv_hbm, zch_hbm,
             agg_hbm,
             srcb, dstb, exb, rowb, invb, wb, outb, sbuf, dbuf, sem0,
             sem1, acc_s):
    cid = lax.axis_index("c")
    sid = lax.axis_index("s")
    rstart = sid * NROWS_T
    pltpu.sync_copy(zch_hbm.at[pl.ds(rstart, NROWS_T)],
                    acc_s.at[pl.ds(rstart, NROWS_T)])

    plsc.subcore_barrier()

    iota = lax.iota(jnp.int32, 16)

    @pl.loop(0, NB3)
    def _(b):
        bb = sid * ECH_T + b * EB
        pltpu.sync_copy(src_hbm.at[pl.ds(bb, EB)], srcb)
        pltpu.sync_copy(dst_hbm.at[pl.ds(bb, EB)], dstb)
        for h in range(H):
            pltpu.sync_copy(ex_hbm.at[pl.ds(h * EPAD + bb, EB)],
                            exb.at[pl.ds(h * EB, EB)])

        @pl.loop(0, NGB)
        def _(g):
            sbuf[...] = srcb[pl.ds(g * 16, 16)] * 2 + cid
            dbuf[...] = dstb[pl.ds(g * 16, 16)]
            cpa = pltpu.async_copy(xl2_hbm.at[sbuf], rowb, sem0)
            cpb = pltpu.async_copy(inv_hbm.at[dbuf], invb, sem1)
            cpa.wait()
            cpb.wait()
            for h in range(H):
                hv = jnp.full((16,), h, jnp.int32)
                exh = exb[pl.ds(h * EB + g * 16, 16)]
                invh = plsc.load_gather(invb, [iota, hv])
                plsc.store_scatter(wb, [iota, hv], exh * invh * 0.25)

            @pl.loop(0, 16)
            def _(j):
                jv = jnp.full((16,), j, jnp.int32)
                ws = [plsc.load_gather(wb,
                                       [jv, jnp.full((16,), h, jnp.int32)])
                      for h in range(H)]
                for c8 in range(CH // 16):
                    cv = c8 * 16 + iota
                    acc = ws[0] * plsc.load_gather(
                        rowb, [jv, jnp.full((16,), 0, jnp.int32), cv])
                    for h in range(1, H):
                        acc = acc + ws[h] * plsc.load_gather(
                            rowb, [jv, jnp.full((16,), h, jnp.int32), cv])
                    plsc.store_scatter(outb, [jv, cv], acc)

            pltpu.sync_copy(outb, acc_s.at[dbuf], add=True)

    plsc.subcore_barrier()
    pltpu.sync_copy(acc_s.at[pl.ds(rstart, NROWS_T)],
                    agg_hbm.at[pl.ds(cid * NPAD + rstart, NROWS_T)])


def _p3(xl2, src, dst, ex, inv, zch):
    mesh = plsc.VectorSubcoreMesh(core_axis_name="c", subcore_axis_name="s")
    k = pl.kernel(
        _p3_body,
        out_type=jax.ShapeDtypeStruct((2 * NPAD, 128), jnp.float32),
        mesh=mesh,
        scratch_types=[
            pltpu.VMEM((EB,), jnp.int32),
            pltpu.VMEM((EB,), jnp.int32),
            pltpu.VMEM((H * EB,), jnp.float32),
            pltpu.VMEM((16, 4, 128), jnp.float32),
            pltpu.VMEM((16, 128), jnp.float32),
            pltpu.VMEM((16, 16), jnp.float32),
            pltpu.VMEM((16, 128), jnp.float32),
            pltpu.VMEM((16,), jnp.int32),
            pltpu.VMEM((16,), jnp.int32),
            pltpu.SemaphoreType.DMA,
            pltpu.SemaphoreType.DMA,
            pltpu.VMEM_SHARED((NPAD, 128), jnp.float32),
        ],
        compiler_params=_sc_params(),
    )
    return k(xl2, src, dst, ex, inv, zch)


def _post_body(agg_ref, res_ref, cb_ref, g_ref, b_ref, Wout_ref, bout_ref,
               out_ref):
    x = (jnp.concatenate([agg_ref[0], agg_ref[1]], axis=-1)
         + res_ref[...] + cb_ref[...])
    mu = jnp.mean(x, axis=-1, keepdims=True)
    var = jnp.mean((x - mu) ** 2, axis=-1, keepdims=True)
    xn = (x - mu) * jax.lax.rsqrt(var + 1e-5) * g_ref[...] + b_ref[...]
    act = _leaky(xn, 0.01)
    out_ref[...] = (
        jnp.dot(act, Wout_ref[...], preferred_element_type=jnp.float32)
        + bout_ref[...]
    )


def _post(agg, res, conv_bias, ln_gamma, ln_beta, Wout, bout):
    return pl.pallas_call(
        _post_body,
        grid=(NPAD // BR,),
        in_specs=[
            pl.BlockSpec((2, BR, CH), lambda i: (0, i, 0)),
            pl.BlockSpec((BR, C), lambda i: (i, 0)),
            pl.BlockSpec((1, C), lambda i: (0, 0)),
            pl.BlockSpec((1, C), lambda i: (0, 0)),
            pl.BlockSpec((1, C), lambda i: (0, 0)),
            pl.BlockSpec((C, C), lambda i: (0, 0)),
            pl.BlockSpec((1, C), lambda i: (0, 0)),
        ],
        out_specs=pl.BlockSpec((BR, C), lambda i: (i, 0)),
        out_shape=jax.ShapeDtypeStruct((NPAD, C), jnp.float32),
    )(agg, res, conv_bias, ln_gamma, ln_beta, Wout, bout)


def kernel(device_embeddings, candidate_embedding, edge_index, Wl, bl, Wr, br,
           att, Wres, conv_bias, ln_gamma, ln_beta, Wout, bout):
    Wl_p = Wl.reshape(D, H, 2, CH).transpose(0, 2, 1, 3).reshape(D, HC)
    Wr_p = Wr.reshape(D, H, 2, CH).transpose(0, 2, 1, 3).reshape(D, HC)
    bl_p = bl.reshape(H, 2, CH).transpose(1, 0, 2).reshape(HC)
    br_p = br.reshape(H, 2, CH).transpose(1, 0, 2).reshape(HC)
    att_p = att.reshape(H, 2, CH).transpose(1, 0, 2).reshape(HC)

    dev_p = jnp.pad(device_embeddings, ((0, NPAD - N_DEV), (0, 0)))
    cand_p = jnp.pad(candidate_embedding, ((0, NPAD - N_CAND), (0, 0)))
    xl, xr, res = _proj(dev_p, cand_p, Wl_p, bl_p[None, :], Wr_p,
                        br_p[None, :], Wres)

    pad_idx = jnp.full((EPAD - E,), NPAD - 1, jnp.int32)
    src = jnp.concatenate([edge_index[0], pad_idx])
    dst = jnp.concatenate([edge_index[1], pad_idx])

    zch = jnp.zeros((NPAD, 128), jnp.float32)

    ex, den = _p2(xl.reshape(NPAD, 8, 128), xr.reshape(NPAD, 8, 128),
                  src, dst, att_p, zch)
    inv = _inv(den)
    agg = _p3(xl.reshape(NPAD * 2, 4, 128), src, dst, ex, inv, zch)
    agg = agg.reshape(2, NPAD, CH)

    out = _post(agg, res, conv_bias[None, :], ln_gamma[None, :],
                ln_beta[None, :], Wout, bout[None, :])
    return out[:N_CAND]
